# baseline (speedup 1.0000x reference)
import jax
import jax.numpy as jnp
from jax.experimental import pallas as pl

NU = 25000
NI = 25000
NN = NU + NI
NE = 400000
D = 64
NNE = NN + NE


def _dot_kernel(gu_ref, gi_ref, out_ref):
    out_ref[:] = jnp.sum(gu_ref[:] * gi_ref[:], axis=2)[:, None, :]


def _gcn(x, ei, W, b, n):
    src = ei[0]
    dst = ei[1]
    deg = jnp.zeros((n,), x.dtype).at[dst].add(1.0) + 1.0
    dinv = jax.lax.rsqrt(deg)
    h = x @ W
    norm = dinv[src] * dinv[dst]
    msg = jnp.take(h, src, axis=0) * norm[:, None]
    out = jnp.zeros((n, W.shape[1]), x.dtype).at[dst].add(msg)
    out = out + h * (1.0 / deg)[:, None]
    return out + b


def kernel(Gu, Gi, Ge, Wpn0, bpn0, Wpn1, bpn1, Wpe0, bpe0, Wpe1, bpe1,
           Wnn0, bnn0, Wnn1, bnn1, Wee0, bee0, Wee1, bee1, Wne0, bne0,
           Wne1, bne1, edge_index, node_edge_index, edge_edge_index):
    nn_emb = jnp.concatenate([Gu, Gi], axis=0)
    ee_emb = Ge
    nn_proj = jax.nn.relu(jax.nn.relu(nn_emb @ Wpn0 + bpn0) @ Wpn1 + bpn1)
    ee_proj = jax.nn.relu(jax.nn.relu(ee_emb @ Wpe0 + bpe0) @ Wpe1 + bpe1)
    ne_emb = jnp.concatenate([nn_proj, ee_proj], axis=0)
    Wnn = [(Wnn0, bnn0), (Wnn1, bnn1)]
    Wee = [(Wee0, bee0), (Wee1, bee1)]
    Wne = [(Wne0, bne0), (Wne1, bne1)]
    for l in range(2):
        nn_emb = _gcn(nn_emb, edge_index, Wnn[l][0], Wnn[l][1], NN)
        ee_emb = _gcn(ee_emb, edge_edge_index, Wee[l][0], Wee[l][1], NE)
        ne_emb = _gcn(ne_emb, node_edge_index, Wne[l][0], Wne[l][1], NNE)
        ne_node = ne_emb[:NN]
        ne_edge = ne_emb[NN:]
        nn_emb = jnp.concatenate([nn_emb, ne_node], axis=1)
        ee_emb = jnp.concatenate([ee_emb, ne_edge], axis=1)
        ne_emb = jnp.concatenate([nn_emb, ee_emb], axis=0)
    gu = nn_emb[:NU]
    gi = nn_emb[NU:]
    gu3 = gu.reshape(25, 1000, 2 * D)
    gi3 = gi.reshape(25, 1000, 2 * D)
    xui = pl.pallas_call(
        _dot_kernel,
        grid=(25,),
        in_specs=[
            pl.BlockSpec((1, 1000, 2 * D), lambda i: (i, 0, 0)),
            pl.BlockSpec((1, 1000, 2 * D), lambda i: (i, 0, 0)),
        ],
        out_specs=pl.BlockSpec((1, 1, 1000), lambda i: (i, 0, 0)),
        out_shape=jax.ShapeDtypeStruct((25, 1, 1000), jnp.float32),
    )(gu3, gi3)
    return xui.reshape(NU)


# speedup vs baseline: 4.6714x; 4.6714x over previous
import functools

import jax
import jax.numpy as jnp
from jax import lax
from jax.experimental import pallas as pl
from jax.experimental.pallas import tpu as pltpu
from jax.experimental.pallas import tpu_sc as plsc

NU = 25000
NI = 25000
NN = NU + NI
NE = 400000
D = 64
NNE = NN + NE

NC = 2
NS = 16
NW = NC * NS

DEG_N = NN + NE + NNE
DEG_SH = 901120
DEG_SLICE = DEG_SH // NS
E_TOT = 3200000
DEG_ROWS = 25600
DEG_RPW = DEG_ROWS // NW
DEG_ITERS = DEG_RPW // 8


def _deg_body(dst_hbm, zz_hbm, out_hbm, idx_v, ones_v, shared):
    c = lax.axis_index("c")
    s = lax.axis_index("s")
    wid = s * NC + c
    for i in range(8):
        ones_v[pl.ds(i * 16, 16)] = jnp.full((16,), 1.0, jnp.float32)
    zoff = pl.multiple_of(s * DEG_SLICE, 1024)
    pltpu.sync_copy(zz_hbm.at[pl.ds(zoff, DEG_SLICE)],
                    shared.at[pl.ds(zoff, DEG_SLICE)])
    plsc.subcore_barrier()

    def step(it, carry):
        row0 = wid * DEG_RPW + it * 8
        pltpu.sync_copy(dst_hbm.at[pl.ds(row0, 8)], idx_v)
        for j in range(8):
            pltpu.sync_copy(ones_v, shared.at[idx_v.at[j]], add=True)
        return carry

    lax.fori_loop(0, DEG_ITERS, step, 0)
    plsc.subcore_barrier()
    pltpu.sync_copy(shared.at[pl.ds(zoff, DEG_SLICE)],
                    out_hbm.at[pl.ds(pl.multiple_of(c * DEG_SH + s * DEG_SLICE, 1024), DEG_SLICE)])


def _sc_degree(dst_all_2d, zz):
    mesh = plsc.VectorSubcoreMesh(core_axis_name="c", subcore_axis_name="s")
    f = pl.kernel(
        _deg_body,
        out_type=jax.ShapeDtypeStruct((NC * DEG_SH,), jnp.float32),
        mesh=mesh,
        scratch_types=[
            pltpu.VMEM((8, 128), jnp.int32),
            pltpu.VMEM((128,), jnp.float32),
            pltpu.VMEM_SHARED((DEG_SH,), jnp.float32),
        ],
    )
    return f(dst_all_2d, zz)


def _dot_kernel(gu_ref, gi_ref, out_ref):
    out_ref[:] = jnp.sum(gu_ref[:] * gi_ref[:], axis=2)[:, None, :]


def _proj_body(x_ref, w0_ref, b0_ref, w1_ref, b1_ref, out_ref):
    h = jnp.maximum(jnp.dot(x_ref[:], w0_ref[:],
                            preferred_element_type=jnp.float32) + b0_ref[:], 0.0)
    out_ref[:] = jnp.maximum(jnp.dot(h, w1_ref[:],
                                     preferred_element_type=jnp.float32)
                             + b1_ref[:], 0.0)


def _tc_proj(x, W0, b0, W1, b1, blk=2000):
    n = x.shape[0]
    din = x.shape[1]
    return pl.pallas_call(
        _proj_body,
        grid=(n // blk,),
        in_specs=[
            pl.BlockSpec((blk, din), lambda i: (i, 0)),
            pl.BlockSpec((din, D), lambda i: (0, 0)),
            pl.BlockSpec((D,), lambda i: (0,)),
            pl.BlockSpec((D, D), lambda i: (0, 0)),
            pl.BlockSpec((D,), lambda i: (0,)),
        ],
        out_specs=pl.BlockSpec((blk, D), lambda i: (i, 0)),
        out_shape=jax.ShapeDtypeStruct((n, D), jnp.float32),
    )(x, W0, b0, W1, b1)


def _pre_body(x_ref, w_ref, deg_ref, h_ref, hs_ref):
    h = jnp.dot(x_ref[:], w_ref[:], preferred_element_type=jnp.float32)
    h_ref[:] = h
    hs_ref[:] = h * jax.lax.rsqrt(deg_ref[:])


def _tc_pre(x, W, deg, blk=2000):
    n = x.shape[0]
    din = x.shape[1]
    h, hs = pl.pallas_call(
        _pre_body,
        grid=(n // blk,),
        in_specs=[
            pl.BlockSpec((blk, din), lambda i: (i, 0)),
            pl.BlockSpec((din, D), lambda i: (0, 0)),
            pl.BlockSpec((blk, 1), lambda i: (i, 0)),
        ],
        out_specs=[
            pl.BlockSpec((blk, D), lambda i: (i, 0)),
            pl.BlockSpec((blk, D), lambda i: (i, 0)),
        ],
        out_shape=[
            jax.ShapeDtypeStruct((n, D), jnp.float32),
            jax.ShapeDtypeStruct((n, D), jnp.float32),
        ],
    )(x, W, deg.reshape(n, 1))
    return h, hs


def _post_body(scat_ref, h_ref, deg_ref, b_ref, out_ref):
    deg = deg_ref[:]
    out_ref[:] = (scat_ref[:] * jax.lax.rsqrt(deg) + h_ref[:] / deg
                  + b_ref[:])


def _tc_post(scat, h, deg, b, blk=2000):
    n = h.shape[0]
    return pl.pallas_call(
        _post_body,
        grid=(n // blk,),
        in_specs=[
            pl.BlockSpec((blk, D), lambda i: (i, 0)),
            pl.BlockSpec((blk, D), lambda i: (i, 0)),
            pl.BlockSpec((blk, 1), lambda i: (i, 0)),
            pl.BlockSpec((D,), lambda i: (0,)),
        ],
        out_specs=pl.BlockSpec((blk, D), lambda i: (i, 0)),
        out_shape=jax.ShapeDtypeStruct((n, D), jnp.float32),
    )(scat, h, deg.reshape(n, 1), b)


CH = 16384
CHB = 14
CHT = CH // NS


def _cnt_body(nbg, nblk, dst_hbm, out_hbm, dbuf, hist):
    c = lax.axis_index("c")
    s = lax.axis_index("s")
    wid = s * NC + c
    zero = jnp.zeros((16,), jnp.int32)
    one = jnp.full((16,), 1, jnp.int32)
    for b in range(nbg):
        hist[pl.ds(b * 16, 16)] = zero

    def grp16(i, carry):
        bv = dbuf[pl.ds(i * 16, 16)] >> CHB
        for j in range(16):
            b16 = bv[j] * 16
            hist[pl.ds(b16, 16)] = hist[pl.ds(b16, 16)] + one
        return carry

    def blk(bi, carry):
        e0 = pl.multiple_of((wid * nblk + bi) * 512, 512)
        pltpu.sync_copy(dst_hbm.at[pl.ds(e0, 512)], dbuf)
        return lax.fori_loop(0, 32, grp16, carry)

    lax.fori_loop(0, nblk, blk, 0)
    pltpu.sync_copy(hist, out_hbm.at[pl.ds(pl.multiple_of(wid * nbg * 16, 16), nbg * 16)])


def _bin_body(nbg, nblk, offp, src_hbm, dst_hbm, offs_hbm,
              bsrc_hbm, bdst_hbm, sbuf, dbuf, offs_v, stg_s, stg_d,
              fill, pos):
    c = lax.axis_index("c")
    s = lax.axis_index("s")
    wid = s * NC + c
    zero = jnp.zeros((16,), jnp.int32)
    lanes = lax.iota(jnp.int32, 16)
    pltpu.sync_copy(offs_hbm, offs_v)
    for b in range(nbg):
        fill[pl.ds(b * 16, 16)] = zero
        pv = offs_v[pl.ds(b * 32 + wid, 16)]
        pos[pl.ds(b * 16, 16)] = jnp.full((16,), 1, jnp.int32) * pv[0]

    def grp16(i, carry):
            dv = dbuf[pl.ds(i * 16, 16)]
            sv = sbuf[pl.ds(i * 16, 16)]
            bv = dv >> CHB
            dl = dv & (CH - 1)
            for j in range(16):
                b16 = bv[j] * 16
                b144 = bv[j] * 144
                f = fill[pl.ds(b16, 16)][0]
                stg_s[pl.ds(b144 + f, 16)] = jnp.full((16,), 1, jnp.int32) * sv[j]
                stg_d[pl.ds(b144 + f, 16)] = jnp.full((16,), 1, jnp.int32) * dl[j]
                fnew = f + 1

                @pl.when(fnew == 128)
                def _fl():
                    p = pl.multiple_of(pos[pl.ds(b16, 16)][0], 128)
                    pltpu.sync_copy(stg_s.at[pl.ds(b144, 128)],
                                    bsrc_hbm.at[pl.ds(p, 128)])
                    pltpu.sync_copy(stg_d.at[pl.ds(b144, 128)],
                                    bdst_hbm.at[pl.ds(p, 128)])
                    pos[pl.ds(b16, 16)] = jnp.full((16,), 1, jnp.int32) * (p + 128)

                fill[pl.ds(b16, 16)] = (jnp.full((16,), 1, jnp.int32)
                                        * jnp.where(fnew == 128, 0, fnew))
            return carry

    def blk(bi, carry):
        e0 = pl.multiple_of((wid * nblk + bi) * 512, 512)
        pltpu.sync_copy(src_hbm.at[pl.ds(e0, 512)], sbuf)
        pltpu.sync_copy(dst_hbm.at[pl.ds(e0, 512)], dbuf)
        return lax.fori_loop(0, 32, grp16, carry)

    lax.fori_loop(0, nblk, blk, 0)
    for b in range(nbg):
        f = fill[pl.ds(b * 16, 16)][0]

        @pl.when(f > 0)
        def _tail():
            for i in range(8):
                ln = i * 16 + lanes
                cs = stg_s[pl.ds(b * 144 + i * 16, 16)]
                cd = stg_d[pl.ds(b * 144 + i * 16, 16)]
                keep = ln < f
                stg_s[pl.ds(b * 144 + i * 16, 16)] = jnp.where(keep, cs, 0)
                stg_d[pl.ds(b * 144 + i * 16, 16)] = jnp.where(keep, cd, CH)
            p = pl.multiple_of(pos[pl.ds(b * 16, 16)][0], 128)
            pltpu.sync_copy(stg_s.at[pl.ds(b * 144, 128)],
                            bsrc_hbm.at[pl.ds(p, 128)])
            pltpu.sync_copy(stg_d.at[pl.ds(b * 144, 128)],
                            bdst_hbm.at[pl.ds(p, 128)])


def _scat2_body(npass, offp, hs_hbm, bsrc_hbm, bdst_hbm, offs_hbm, zz_hbm,
                out_hbm, offs_v, gsrc, gdst, rows_v, sem, shared):
    c = lax.axis_index("c")
    s = lax.axis_index("s")
    pltpu.sync_copy(offs_hbm, offs_v)

    def do_pass(p, carry):
        b = p * NC + c
        lo = b * CH
        rs = pl.multiple_of(offs_v[pl.ds(b * 32, 16)][0], 128)
        re = offs_v[pl.ds((b + 1) * 32, 16)][0]
        ng = (re - rs) >> 7
        woff = pl.multiple_of(s * CHT, 1024)
        pltpu.sync_copy(zz_hbm.at[pl.ds(woff, CHT)],
                        shared.at[pl.ds(woff, CHT)])
        plsc.subcore_barrier()
        g0 = (ng * s) >> 4
        g1 = (ng * (s + 1)) >> 4

        def grp(g, cc):
            base = pl.multiple_of(rs + g * 128, 128)
            pltpu.sync_copy(bsrc_hbm.at[pl.ds(base, 128)], gsrc)
            pltpu.sync_copy(bdst_hbm.at[pl.ds(base, 128)], gdst)
            pltpu.async_copy(hs_hbm.at[gsrc], rows_v, sem).wait()
            pltpu.sync_copy(rows_v, shared.at[gdst], add=True)
            return cc

        lax.fori_loop(g0, g1, grp, 0)
        plsc.subcore_barrier()
        pltpu.sync_copy(shared.at[pl.ds(woff, CHT)],
                        out_hbm.at[pl.ds(pl.multiple_of(lo + s * CHT, 1024), CHT)])
        plsc.subcore_barrier()
        return carry

    lax.fori_loop(0, npass, do_pass, 0)


def _mesh():
    return plsc.VectorSubcoreMesh(core_axis_name="c", subcore_axis_name="s")


def _graph_params(E, n):
    epad = -(-E // 16384) * 16384
    npass = -(-n // (NC * CH))
    nbg = npass * NC
    npad = nbg * CH
    sz = epad + 32 * nbg * 128
    offp = -(-(nbg * 32 + 16) // 8) * 8
    return epad, npass, nbg, npad, sz, offp


def _sc_prep(src, dst, E, n):
    epad, npass, nbg, npad, sz, offp = _graph_params(E, n)
    if epad != E:
        pad = epad - E
        src = jnp.concatenate([src, jnp.zeros((pad,), jnp.int32)])
        dst = jnp.concatenate([dst, jnp.full((pad,), npad - 1, jnp.int32)])
    nblk = epad // (32 * 512)
    cnt = pl.kernel(
        functools.partial(_cnt_body, nbg, nblk),
        out_type=jax.ShapeDtypeStruct((32 * nbg * 16,), jnp.int32),
        mesh=_mesh(),
        scratch_types=[
            pltpu.VMEM((512,), jnp.int32),
            pltpu.VMEM((nbg * 16,), jnp.int32),
        ],
    )(dst)
    counts = cnt.reshape(32, nbg, 16)[:, :, 0]
    pc = ((counts + 127) // 128) * 128
    flat = pc.T.reshape(-1)
    offs = jnp.concatenate([jnp.zeros((1,), jnp.int32),
                            jnp.cumsum(flat, dtype=jnp.int32)])
    offs = jnp.concatenate(
        [offs, jnp.full((offp - nbg * 32 - 1,), offs[-1], jnp.int32)])
    bsrc, bdst = pl.kernel(
        functools.partial(_bin_body, nbg, nblk, offp),
        out_type=[jax.ShapeDtypeStruct((sz,), jnp.int32),
                  jax.ShapeDtypeStruct((sz,), jnp.int32)],
        mesh=_mesh(),
        scratch_types=[
            pltpu.VMEM((512,), jnp.int32),
            pltpu.VMEM((512,), jnp.int32),
            pltpu.VMEM((offp,), jnp.int32),
            pltpu.VMEM((nbg * 144,), jnp.int32),
            pltpu.VMEM((nbg * 144,), jnp.int32),
            pltpu.VMEM((nbg * 16,), jnp.int32),
            pltpu.VMEM((nbg * 16,), jnp.int32),
        ],
    )(src, dst, offs)
    return bsrc, bdst, offs


def _sc_scatter2(hs, prep, E, n):
    bsrc, bdst, offs = prep
    epad, npass, nbg, npad, sz, offp = _graph_params(E, n)
    zz = jnp.zeros((CH, D), jnp.float32)
    out = pl.kernel(
        functools.partial(_scat2_body, npass, offp),
        out_type=jax.ShapeDtypeStruct((npad, D), jnp.float32),
        mesh=_mesh(),
        compiler_params=pltpu.CompilerParams(use_tc_tiling_on_sc=False),
        scratch_types=[
            pltpu.VMEM((offp,), jnp.int32),
            pltpu.VMEM((128,), jnp.int32),
            pltpu.VMEM((128,), jnp.int32),
            pltpu.VMEM((128, D), jnp.float32),
            pltpu.SemaphoreType.DMA,
            pltpu.VMEM_SHARED((CH + 8, D), jnp.float32),
        ],
    )(hs, bsrc, bdst, offs, zz)
    return out[:n]


def _gcn(x, prep, E, W, b, n, deg):
    h, hs = _tc_pre(x, W, deg)
    scat = _sc_scatter2(hs, prep, E, n)
    return _tc_post(scat, h, deg, b)


def kernel(Gu, Gi, Ge, Wpn0, bpn0, Wpn1, bpn1, Wpe0, bpe0, Wpe1, bpe1,
           Wnn0, bnn0, Wnn1, bnn1, Wee0, bee0, Wee1, bee1, Wne0, bne0,
           Wne1, bne1, edge_index, node_edge_index, edge_edge_index):
    nn_emb = jnp.concatenate([Gu, Gi], axis=0)
    ee_emb = Ge

    dst_all = jnp.concatenate([
        edge_index[1],
        edge_edge_index[1] + NN,
        node_edge_index[1] + (NN + NE),
        jnp.full((DEG_ROWS * 128 - E_TOT,), DEG_N, jnp.int32),
    ]).reshape(DEG_ROWS, 128)
    zz = jnp.zeros((DEG_SH,), jnp.float32)
    parts = _sc_degree(dst_all, zz).reshape(NC, DEG_SH)
    hist = parts[0] + parts[1]
    deg_nn = hist[0:NN] + 1.0
    deg_ee = hist[NN:NN + NE] + 1.0
    deg_ne = hist[NN + NE:NN + NE + NNE] + 1.0

    nn_proj = _tc_proj(nn_emb, Wpn0, bpn0, Wpn1, bpn1)
    ee_proj = _tc_proj(ee_emb, Wpe0, bpe0, Wpe1, bpe1)
    ne_emb = jnp.concatenate([nn_proj, ee_proj], axis=0)
    Wnn = [(Wnn0, bnn0), (Wnn1, bnn1)]
    Wee = [(Wee0, bee0), (Wee1, bee1)]
    Wne = [(Wne0, bne0), (Wne1, bne1)]
    prep_nn = _sc_prep(edge_index[0], edge_index[1], 800000, NN)
    prep_ee = _sc_prep(edge_edge_index[0], edge_edge_index[1], 800000, NE)
    prep_ne = _sc_prep(node_edge_index[0], node_edge_index[1], 1600000, NNE)
    for l in range(2):
        nn_emb = _gcn(nn_emb, prep_nn, 800000, Wnn[l][0], Wnn[l][1], NN, deg_nn)
        ee_emb = _gcn(ee_emb, prep_ee, 800000, Wee[l][0], Wee[l][1], NE, deg_ee)
        ne_emb = _gcn(ne_emb, prep_ne, 1600000, Wne[l][0], Wne[l][1], NNE, deg_ne)
        ne_node = ne_emb[:NN]
        ne_edge = ne_emb[NN:]
        nn_emb = jnp.concatenate([nn_emb, ne_node], axis=1)
        ee_emb = jnp.concatenate([ee_emb, ne_edge], axis=1)
        ne_emb = jnp.concatenate([nn_emb, ee_emb], axis=0)
    gu = nn_emb[:NU]
    gi = nn_emb[NU:]
    gu3 = gu.reshape(25, 1000, 2 * D)
    gi3 = gi.reshape(25, 1000, 2 * D)
    xui = pl.pallas_call(
        _dot_kernel,
        grid=(25,),
        in_specs=[
            pl.BlockSpec((1, 1000, 2 * D), lambda i: (i, 0, 0)),
            pl.BlockSpec((1, 1000, 2 * D), lambda i: (i, 0, 0)),
        ],
        out_specs=pl.BlockSpec((1, 1, 1000), lambda i: (i, 0, 0)),
        out_shape=jax.ShapeDtypeStruct((25, 1, 1000), jnp.float32),
    )(gu3, gi3)
    return xui.reshape(NU)


# speedup vs baseline: 4.8864x; 1.0460x over previous
import functools

import jax
import jax.numpy as jnp
from jax import lax
from jax.experimental import pallas as pl
from jax.experimental.pallas import tpu as pltpu
from jax.experimental.pallas import tpu_sc as plsc

NU = 25000
NI = 25000
NN = NU + NI
NE = 400000
D = 64
NNE = NN + NE

NC = 2
NS = 16
NW = NC * NS

DEG_N = NN + NE + NNE
DEG_SH = 901120
DEG_SLICE = DEG_SH // NS
E_TOT = 3200000
DEG_ROWS = 25600
DEG_RPW = DEG_ROWS // NW
DEG_ITERS = DEG_RPW // 8


def _deg_body(dst_hbm, zz_hbm, out_hbm, idx_v, ones_v, semD, shared):
    c = lax.axis_index("c")
    s = lax.axis_index("s")
    wid = s * NC + c
    for i in range(8):
        ones_v[pl.ds(i * 16, 16)] = jnp.full((16,), 1.0, jnp.float32)
    zoff = pl.multiple_of(s * DEG_SLICE, 1024)
    pltpu.sync_copy(zz_hbm.at[pl.ds(zoff, DEG_SLICE)],
                    shared.at[pl.ds(zoff, DEG_SLICE)])
    plsc.subcore_barrier()

    def step(it, carry):
        row0 = wid * DEG_RPW + it * 8
        pltpu.sync_copy(dst_hbm.at[pl.ds(row0, 8)], idx_v)
        for j in range(8):
            pltpu.async_copy(ones_v, shared.at[idx_v.at[j]], semD, add=True)
        for j in range(8):
            pltpu.make_async_copy(zz_hbm.at[pl.ds(0, 128)], ones_v, semD).wait()
        return carry

    lax.fori_loop(0, DEG_ITERS, step, 0)
    plsc.subcore_barrier()
    pltpu.sync_copy(shared.at[pl.ds(zoff, DEG_SLICE)],
                    out_hbm.at[pl.ds(pl.multiple_of(c * DEG_SH + s * DEG_SLICE, 1024), DEG_SLICE)])


def _sc_degree(dst_all_2d, zz):
    mesh = plsc.VectorSubcoreMesh(core_axis_name="c", subcore_axis_name="s")
    f = pl.kernel(
        _deg_body,
        out_type=jax.ShapeDtypeStruct((NC * DEG_SH,), jnp.float32),
        mesh=mesh,
        scratch_types=[
            pltpu.VMEM((8, 128), jnp.int32),
            pltpu.VMEM((128,), jnp.float32),
            pltpu.SemaphoreType.DMA,
            pltpu.VMEM_SHARED((DEG_SH,), jnp.float32),
        ],
    )
    return f(dst_all_2d, zz)


def _dot_kernel(gu_ref, gi_ref, out_ref):
    out_ref[:] = jnp.sum(gu_ref[:] * gi_ref[:], axis=2)[:, None, :]


def _proj_body(x_ref, w0_ref, b0_ref, w1_ref, b1_ref, out_ref):
    h = jnp.maximum(jnp.dot(x_ref[:], w0_ref[:],
                            preferred_element_type=jnp.float32) + b0_ref[:], 0.0)
    out_ref[:] = jnp.maximum(jnp.dot(h, w1_ref[:],
                                     preferred_element_type=jnp.float32)
                             + b1_ref[:], 0.0)


def _tc_proj(x, W0, b0, W1, b1, blk=2000):
    n = x.shape[0]
    din = x.shape[1]
    return pl.pallas_call(
        _proj_body,
        grid=(n // blk,),
        in_specs=[
            pl.BlockSpec((blk, din), lambda i: (i, 0)),
            pl.BlockSpec((din, D), lambda i: (0, 0)),
            pl.BlockSpec((D,), lambda i: (0,)),
            pl.BlockSpec((D, D), lambda i: (0, 0)),
            pl.BlockSpec((D,), lambda i: (0,)),
        ],
        out_specs=pl.BlockSpec((blk, D), lambda i: (i, 0)),
        out_shape=jax.ShapeDtypeStruct((n, D), jnp.float32),
    )(x, W0, b0, W1, b1)


def _pre_body(x_ref, w_ref, deg_ref, h_ref, hs_ref):
    h = jnp.dot(x_ref[:], w_ref[:], preferred_element_type=jnp.float32)
    h_ref[:] = h
    hs_ref[:] = h * jax.lax.rsqrt(deg_ref[:])


def _tc_pre(x, W, deg, blk=2000):
    n = x.shape[0]
    din = x.shape[1]
    h, hs = pl.pallas_call(
        _pre_body,
        grid=(n // blk,),
        in_specs=[
            pl.BlockSpec((blk, din), lambda i: (i, 0)),
            pl.BlockSpec((din, D), lambda i: (0, 0)),
            pl.BlockSpec((blk, 1), lambda i: (i, 0)),
        ],
        out_specs=[
            pl.BlockSpec((blk, D), lambda i: (i, 0)),
            pl.BlockSpec((blk, D), lambda i: (i, 0)),
        ],
        out_shape=[
            jax.ShapeDtypeStruct((n, D), jnp.float32),
            jax.ShapeDtypeStruct((n, D), jnp.float32),
        ],
    )(x, W, deg.reshape(n, 1))
    return h, hs


def _post_body(scat_ref, h_ref, deg_ref, b_ref, out_ref):
    deg = deg_ref[:]
    out_ref[:] = (scat_ref[:] * jax.lax.rsqrt(deg) + h_ref[:] / deg
                  + b_ref[:])


def _tc_post(scat, h, deg, b, blk=2000):
    n = h.shape[0]
    return pl.pallas_call(
        _post_body,
        grid=(n // blk,),
        in_specs=[
            pl.BlockSpec((blk, D), lambda i: (i, 0)),
            pl.BlockSpec((blk, D), lambda i: (i, 0)),
            pl.BlockSpec((blk, 1), lambda i: (i, 0)),
            pl.BlockSpec((D,), lambda i: (0,)),
        ],
        out_specs=pl.BlockSpec((blk, D), lambda i: (i, 0)),
        out_shape=jax.ShapeDtypeStruct((n, D), jnp.float32),
    )(scat, h, deg.reshape(n, 1), b)


CH = 16384
CHB = 14
CHT = CH // NS


def _cnt_body(nbg, nblk, dst_hbm, out_hbm, dbuf, hist):
    c = lax.axis_index("c")
    s = lax.axis_index("s")
    wid = s * NC + c
    zero = jnp.zeros((16,), jnp.int32)
    one = jnp.full((16,), 1, jnp.int32)
    for b in range(nbg):
        hist[pl.ds(b * 16, 16)] = zero

    def grp16(i, carry):
        bv = dbuf[pl.ds(i * 16, 16)] >> CHB
        for j in range(16):
            b16 = bv[j] * 16
            hist[pl.ds(b16, 16)] = hist[pl.ds(b16, 16)] + one
        return carry

    def blk(bi, carry):
        e0 = pl.multiple_of((wid * nblk + bi) * 512, 512)
        pltpu.sync_copy(dst_hbm.at[pl.ds(e0, 512)], dbuf)
        return lax.fori_loop(0, 32, grp16, carry)

    lax.fori_loop(0, nblk, blk, 0)
    pltpu.sync_copy(hist, out_hbm.at[pl.ds(pl.multiple_of(wid * nbg * 16, 16), nbg * 16)])


def _bin_body(nbg, nblk, offp, src_hbm, dst_hbm, offs_hbm,
              bsrc_hbm, bdst_hbm, sbuf, dbuf, offs_v, stg_s, stg_d,
              fill, pos):
    c = lax.axis_index("c")
    s = lax.axis_index("s")
    wid = s * NC + c
    zero = jnp.zeros((16,), jnp.int32)
    lanes = lax.iota(jnp.int32, 16)
    pltpu.sync_copy(offs_hbm, offs_v)
    for b in range(nbg):
        fill[pl.ds(b * 16, 16)] = zero
        pv = offs_v[pl.ds(b * 32 + wid, 16)]
        pos[pl.ds(b * 16, 16)] = jnp.full((16,), 1, jnp.int32) * pv[0]

    def grp16(i, carry):
            dv = dbuf[pl.ds(i * 16, 16)]
            sv = sbuf[pl.ds(i * 16, 16)]
            bv = dv >> CHB
            dl = dv & (CH - 1)
            for j in range(16):
                b16 = bv[j] * 16
                b144 = bv[j] * 144
                f = fill[pl.ds(b16, 16)][0]
                stg_s[pl.ds(b144 + f, 16)] = jnp.full((16,), 1, jnp.int32) * sv[j]
                stg_d[pl.ds(b144 + f, 16)] = jnp.full((16,), 1, jnp.int32) * dl[j]
                fnew = f + 1

                @pl.when(fnew == 128)
                def _fl():
                    p = pl.multiple_of(pos[pl.ds(b16, 16)][0], 128)
                    pltpu.sync_copy(stg_s.at[pl.ds(b144, 128)],
                                    bsrc_hbm.at[pl.ds(p, 128)])
                    pltpu.sync_copy(stg_d.at[pl.ds(b144, 128)],
                                    bdst_hbm.at[pl.ds(p, 128)])
                    pos[pl.ds(b16, 16)] = jnp.full((16,), 1, jnp.int32) * (p + 128)

                fill[pl.ds(b16, 16)] = (jnp.full((16,), 1, jnp.int32)
                                        * jnp.where(fnew == 128, 0, fnew))
            return carry

    def blk(bi, carry):
        e0 = pl.multiple_of((wid * nblk + bi) * 512, 512)
        pltpu.sync_copy(src_hbm.at[pl.ds(e0, 512)], sbuf)
        pltpu.sync_copy(dst_hbm.at[pl.ds(e0, 512)], dbuf)
        return lax.fori_loop(0, 32, grp16, carry)

    lax.fori_loop(0, nblk, blk, 0)
    for b in range(nbg):
        f = fill[pl.ds(b * 16, 16)][0]

        @pl.when(f > 0)
        def _tail():
            for i in range(8):
                ln = i * 16 + lanes
                cs = stg_s[pl.ds(b * 144 + i * 16, 16)]
                cd = stg_d[pl.ds(b * 144 + i * 16, 16)]
                keep = ln < f
                stg_s[pl.ds(b * 144 + i * 16, 16)] = jnp.where(keep, cs, 0)
                stg_d[pl.ds(b * 144 + i * 16, 16)] = jnp.where(keep, cd, CH)
            p = pl.multiple_of(pos[pl.ds(b * 16, 16)][0], 128)
            pltpu.sync_copy(stg_s.at[pl.ds(b * 144, 128)],
                            bsrc_hbm.at[pl.ds(p, 128)])
            pltpu.sync_copy(stg_d.at[pl.ds(b * 144, 128)],
                            bdst_hbm.at[pl.ds(p, 128)])


def _scat2_body(npass, offp, hs_hbm, bsrc_hbm, bdst_hbm, offs_hbm, zz_hbm,
                out_hbm, offs_v, gsrcA, gdstA, gsrcB, gdstB, rowsA, rowsB,
                semIA, semIB, semGA, semGB, shared):
    c = lax.axis_index("c")
    s = lax.axis_index("s")
    pltpu.sync_copy(offs_hbm, offs_v)

    def do_pass(p, carry):
        b = p * NC + c
        lo = b * CH
        rs = pl.multiple_of(offs_v[pl.ds(b * 32, 16)][0], 128)
        re = offs_v[pl.ds((b + 1) * 32, 16)][0]
        ng = (re - rs) >> 7
        woff = pl.multiple_of(s * CHT, 1024)
        pltpu.sync_copy(zz_hbm.at[pl.ds(woff, CHT)],
                        shared.at[pl.ds(woff, CHT)])
        plsc.subcore_barrier()
        g0 = (ng * s) >> 4
        g1 = (ng * (s + 1)) >> 4
        tn = g1 - g0

        def gbase(g):
            gc = jnp.minimum(g, g1 - 1)
            return pl.multiple_of(rs + gc * 128, 128)

        def idx_load(g, gsrc, gdst, sem):
            base = gbase(g)
            pltpu.async_copy(bsrc_hbm.at[pl.ds(base, 128)], gsrc, sem)
            pltpu.async_copy(bdst_hbm.at[pl.ds(base, 128)], gdst, sem)

        def idx_wait(gsrc, gdst, sem):
            pltpu.make_async_copy(bsrc_hbm.at[pl.ds(0, 128)], gsrc, sem).wait()
            pltpu.make_async_copy(bdst_hbm.at[pl.ds(0, 128)], gdst, sem).wait()

        def gather(g, gsrc, rows, sem):
            del g
            pltpu.async_copy(hs_hbm.at[gsrc], rows, sem)

        def gather_wait(rows, sem):
            pltpu.make_async_copy(hs_hbm.at[pl.ds(0, 128)], rows, sem).wait()

        @pl.when(tn > 0)
        def _work():
            idx_load(g0, gsrcA, gdstA, semIA)
            idx_wait(gsrcA, gdstA, semIA)
            gather(g0, gsrcA, rowsA, semGA)
            idx_load(g0 + 1, gsrcB, gdstB, semIB)
            npair = tn >> 1

            def pair(q, cc):
                g = g0 + q * 2
                idx_wait(gsrcB, gdstB, semIB)
                gather(g + 1, gsrcB, rowsB, semGB)
                gather_wait(rowsA, semGA)
                pltpu.sync_copy(rowsA, shared.at[gdstA], add=True)
                idx_load(g + 2, gsrcA, gdstA, semIA)
                idx_wait(gsrcA, gdstA, semIA)
                gather(g + 2, gsrcA, rowsA, semGA)
                gather_wait(rowsB, semGB)
                pltpu.sync_copy(rowsB, shared.at[gdstB], add=True)
                idx_load(g + 3, gsrcB, gdstB, semIB)
                return cc

            lax.fori_loop(0, npair, pair, 0)
            gather_wait(rowsA, semGA)

            @pl.when((tn & 1) == 1)
            def _odd():
                pltpu.sync_copy(rowsA, shared.at[gdstA], add=True)

            idx_wait(gsrcB, gdstB, semIB)

        plsc.subcore_barrier()
        pltpu.sync_copy(shared.at[pl.ds(woff, CHT)],
                        out_hbm.at[pl.ds(pl.multiple_of(lo + s * CHT, 1024), CHT)])
        plsc.subcore_barrier()
        return carry

    lax.fori_loop(0, npass, do_pass, 0)


def _mesh():
    return plsc.VectorSubcoreMesh(core_axis_name="c", subcore_axis_name="s")


def _graph_params(E, n):
    epad = -(-E // 16384) * 16384
    npass = -(-n // (NC * CH))
    nbg = npass * NC
    npad = nbg * CH
    sz = epad + 32 * nbg * 128
    offp = -(-(nbg * 32 + 16) // 8) * 8
    return epad, npass, nbg, npad, sz, offp


def _sc_prep(src, dst, E, n):
    epad, npass, nbg, npad, sz, offp = _graph_params(E, n)
    if epad != E:
        pad = epad - E
        src = jnp.concatenate([src, jnp.zeros((pad,), jnp.int32)])
        dst = jnp.concatenate([dst, jnp.full((pad,), npad - 1, jnp.int32)])
    nblk = epad // (32 * 512)
    cnt = pl.kernel(
        functools.partial(_cnt_body, nbg, nblk),
        out_type=jax.ShapeDtypeStruct((32 * nbg * 16,), jnp.int32),
        mesh=_mesh(),
        scratch_types=[
            pltpu.VMEM((512,), jnp.int32),
            pltpu.VMEM((nbg * 16,), jnp.int32),
        ],
    )(dst)
    counts = cnt.reshape(32, nbg, 16)[:, :, 0]
    pc = ((counts + 127) // 128) * 128
    flat = pc.T.reshape(-1)
    offs = jnp.concatenate([jnp.zeros((1,), jnp.int32),
                            jnp.cumsum(flat, dtype=jnp.int32)])
    offs = jnp.concatenate(
        [offs, jnp.full((offp - nbg * 32 - 1,), offs[-1], jnp.int32)])
    bsrc, bdst = pl.kernel(
        functools.partial(_bin_body, nbg, nblk, offp),
        out_type=[jax.ShapeDtypeStruct((sz,), jnp.int32),
                  jax.ShapeDtypeStruct((sz,), jnp.int32)],
        mesh=_mesh(),
        scratch_types=[
            pltpu.VMEM((512,), jnp.int32),
            pltpu.VMEM((512,), jnp.int32),
            pltpu.VMEM((offp,), jnp.int32),
            pltpu.VMEM((nbg * 144,), jnp.int32),
            pltpu.VMEM((nbg * 144,), jnp.int32),
            pltpu.VMEM((nbg * 16,), jnp.int32),
            pltpu.VMEM((nbg * 16,), jnp.int32),
        ],
    )(src, dst, offs)
    return bsrc, bdst, offs


def _sc_scatter2(hs, prep, E, n):
    bsrc, bdst, offs = prep
    epad, npass, nbg, npad, sz, offp = _graph_params(E, n)
    zz = jnp.zeros((CH, D), jnp.float32)
    out = pl.kernel(
        functools.partial(_scat2_body, npass, offp),
        out_type=jax.ShapeDtypeStruct((npad, D), jnp.float32),
        mesh=_mesh(),
        compiler_params=pltpu.CompilerParams(use_tc_tiling_on_sc=False),
        scratch_types=[
            pltpu.VMEM((offp,), jnp.int32),
            pltpu.VMEM((128,), jnp.int32),
            pltpu.VMEM((128,), jnp.int32),
            pltpu.VMEM((128,), jnp.int32),
            pltpu.VMEM((128,), jnp.int32),
            pltpu.VMEM((128, D), jnp.float32),
            pltpu.VMEM((128, D), jnp.float32),
            pltpu.SemaphoreType.DMA,
            pltpu.SemaphoreType.DMA,
            pltpu.SemaphoreType.DMA,
            pltpu.SemaphoreType.DMA,
            pltpu.VMEM_SHARED((CH + 8, D), jnp.float32),
        ],
    )(hs, bsrc, bdst, offs, zz)
    return out[:n]


def _gcn(x, prep, E, W, b, n, deg):
    h, hs = _tc_pre(x, W, deg)
    scat = _sc_scatter2(hs, prep, E, n)
    return _tc_post(scat, h, deg, b)


def kernel(Gu, Gi, Ge, Wpn0, bpn0, Wpn1, bpn1, Wpe0, bpe0, Wpe1, bpe1,
           Wnn0, bnn0, Wnn1, bnn1, Wee0, bee0, Wee1, bee1, Wne0, bne0,
           Wne1, bne1, edge_index, node_edge_index, edge_edge_index):
    nn_emb = jnp.concatenate([Gu, Gi], axis=0)
    ee_emb = Ge

    dst_all = jnp.concatenate([
        edge_index[1],
        edge_edge_index[1] + NN,
        node_edge_index[1] + (NN + NE),
        jnp.full((DEG_ROWS * 128 - E_TOT,), DEG_N, jnp.int32),
    ]).reshape(DEG_ROWS, 128)
    zz = jnp.zeros((DEG_SH,), jnp.float32)
    parts = _sc_degree(dst_all, zz).reshape(NC, DEG_SH)
    hist = parts[0] + parts[1]
    deg_nn = hist[0:NN] + 1.0
    deg_ee = hist[NN:NN + NE] + 1.0
    deg_ne = hist[NN + NE:NN + NE + NNE] + 1.0

    nn_proj = _tc_proj(nn_emb, Wpn0, bpn0, Wpn1, bpn1)
    ee_proj = _tc_proj(ee_emb, Wpe0, bpe0, Wpe1, bpe1)
    ne_emb = jnp.concatenate([nn_proj, ee_proj], axis=0)
    Wnn = [(Wnn0, bnn0), (Wnn1, bnn1)]
    Wee = [(Wee0, bee0), (Wee1, bee1)]
    Wne = [(Wne0, bne0), (Wne1, bne1)]
    prep_nn = _sc_prep(edge_index[0], edge_index[1], 800000, NN)
    prep_ee = _sc_prep(edge_edge_index[0], edge_edge_index[1], 800000, NE)
    prep_ne = _sc_prep(node_edge_index[0], node_edge_index[1], 1600000, NNE)
    for l in range(2):
        nn_emb = _gcn(nn_emb, prep_nn, 800000, Wnn[l][0], Wnn[l][1], NN, deg_nn)
        ee_emb = _gcn(ee_emb, prep_ee, 800000, Wee[l][0], Wee[l][1], NE, deg_ee)
        ne_emb = _gcn(ne_emb, prep_ne, 1600000, Wne[l][0], Wne[l][1], NNE, deg_ne)
        ne_node = ne_emb[:NN]
        ne_edge = ne_emb[NN:]
        nn_emb = jnp.concatenate([nn_emb, ne_node], axis=1)
        ee_emb = jnp.concatenate([ee_emb, ne_edge], axis=1)
        ne_emb = jnp.concatenate([nn_emb, ee_emb], axis=0)
    gu = nn_emb[:NU]
    gi = nn_emb[NU:]
    gu3 = gu.reshape(25, 1000, 2 * D)
    gi3 = gi.reshape(25, 1000, 2 * D)
    xui = pl.pallas_call(
        _dot_kernel,
        grid=(25,),
        in_specs=[
            pl.BlockSpec((1, 1000, 2 * D), lambda i: (i, 0, 0)),
            pl.BlockSpec((1, 1000, 2 * D), lambda i: (i, 0, 0)),
        ],
        out_specs=pl.BlockSpec((1, 1, 1000), lambda i: (i, 0, 0)),
        out_shape=jax.ShapeDtypeStruct((25, 1, 1000), jnp.float32),
    )(gu3, gi3)
    return xui.reshape(NU)


# speedup vs baseline: 4.9044x; 1.0037x over previous
import functools

import jax
import jax.numpy as jnp
from jax import lax
from jax.experimental import pallas as pl
from jax.experimental.pallas import tpu as pltpu
from jax.experimental.pallas import tpu_sc as plsc

NU = 25000
NI = 25000
NN = NU + NI
NE = 400000
D = 64
NNE = NN + NE

NC = 2
NS = 16
NW = NC * NS

DEG_N = NN + NE + NNE
DEG_SH = 901120
DEG_SLICE = DEG_SH // NS
E_TOT = 3200000
DEG_ROWS = 25600
DEG_RPW = DEG_ROWS // NW
DEG_ITERS = DEG_RPW // 8


def _deg_body(dst_hbm, zz_hbm, out_hbm, idx_v, ones_v, semD, shared):
    c = lax.axis_index("c")
    s = lax.axis_index("s")
    wid = s * NC + c
    for i in range(8):
        ones_v[pl.ds(i * 16, 16)] = jnp.full((16,), 1.0, jnp.float32)
    zoff = pl.multiple_of(s * DEG_SLICE, 1024)
    pltpu.sync_copy(zz_hbm.at[pl.ds(zoff, DEG_SLICE)],
                    shared.at[pl.ds(zoff, DEG_SLICE)])
    plsc.subcore_barrier()

    def step(it, carry):
        row0 = wid * DEG_RPW + it * 8
        pltpu.sync_copy(dst_hbm.at[pl.ds(row0, 8)], idx_v)
        for j in range(8):
            pltpu.async_copy(ones_v, shared.at[idx_v.at[j]], semD, add=True)
        for j in range(8):
            pltpu.make_async_copy(zz_hbm.at[pl.ds(0, 128)], ones_v, semD).wait()
        return carry

    lax.fori_loop(0, DEG_ITERS, step, 0)
    plsc.subcore_barrier()
    pltpu.sync_copy(shared.at[pl.ds(zoff, DEG_SLICE)],
                    out_hbm.at[pl.ds(pl.multiple_of(c * DEG_SH + s * DEG_SLICE, 1024), DEG_SLICE)])


def _sc_degree(dst_all_2d, zz):
    mesh = plsc.VectorSubcoreMesh(core_axis_name="c", subcore_axis_name="s")
    f = pl.kernel(
        _deg_body,
        out_type=jax.ShapeDtypeStruct((NC * DEG_SH,), jnp.float32),
        mesh=mesh,
        scratch_types=[
            pltpu.VMEM((8, 128), jnp.int32),
            pltpu.VMEM((128,), jnp.float32),
            pltpu.SemaphoreType.DMA,
            pltpu.VMEM_SHARED((DEG_SH,), jnp.float32),
        ],
    )
    return f(dst_all_2d, zz)


def _dot_kernel(gu_ref, gi_ref, out_ref):
    out_ref[:] = jnp.sum(gu_ref[:] * gi_ref[:], axis=2)[:, None, :]


def _proj_body(x_ref, w0_ref, b0_ref, w1_ref, b1_ref, out_ref):
    h = jnp.maximum(jnp.dot(x_ref[:], w0_ref[:],
                            preferred_element_type=jnp.float32) + b0_ref[:], 0.0)
    out_ref[:] = jnp.maximum(jnp.dot(h, w1_ref[:],
                                     preferred_element_type=jnp.float32)
                             + b1_ref[:], 0.0)


def _tc_proj(x, W0, b0, W1, b1, blk=2000):
    n = x.shape[0]
    din = x.shape[1]
    return pl.pallas_call(
        _proj_body,
        grid=(n // blk,),
        in_specs=[
            pl.BlockSpec((blk, din), lambda i: (i, 0)),
            pl.BlockSpec((din, D), lambda i: (0, 0)),
            pl.BlockSpec((D,), lambda i: (0,)),
            pl.BlockSpec((D, D), lambda i: (0, 0)),
            pl.BlockSpec((D,), lambda i: (0,)),
        ],
        out_specs=pl.BlockSpec((blk, D), lambda i: (i, 0)),
        out_shape=jax.ShapeDtypeStruct((n, D), jnp.float32),
    )(x, W0, b0, W1, b1)


def _pre_body(x_ref, w_ref, deg_ref, h_ref, hs_ref):
    h = jnp.dot(x_ref[:], w_ref[:], preferred_element_type=jnp.float32)
    h_ref[:] = h
    hs_ref[:] = h * jax.lax.rsqrt(deg_ref[:])


def _tc_pre(x, W, deg, blk=2000):
    n = x.shape[0]
    din = x.shape[1]
    h, hs = pl.pallas_call(
        _pre_body,
        grid=(n // blk,),
        in_specs=[
            pl.BlockSpec((blk, din), lambda i: (i, 0)),
            pl.BlockSpec((din, D), lambda i: (0, 0)),
            pl.BlockSpec((blk, 1), lambda i: (i, 0)),
        ],
        out_specs=[
            pl.BlockSpec((blk, D), lambda i: (i, 0)),
            pl.BlockSpec((blk, D), lambda i: (i, 0)),
        ],
        out_shape=[
            jax.ShapeDtypeStruct((n, D), jnp.float32),
            jax.ShapeDtypeStruct((n, D), jnp.float32),
        ],
    )(x, W, deg.reshape(n, 1))
    return h, hs


def _post_body(scat_ref, h_ref, deg_ref, b_ref, out_ref):
    deg = deg_ref[:]
    out_ref[:] = (scat_ref[:] * jax.lax.rsqrt(deg) + h_ref[:] / deg
                  + b_ref[:])


def _tc_post(scat, h, deg, b, blk=2000):
    n = h.shape[0]
    return pl.pallas_call(
        _post_body,
        grid=(n // blk,),
        in_specs=[
            pl.BlockSpec((blk, D), lambda i: (i, 0)),
            pl.BlockSpec((blk, D), lambda i: (i, 0)),
            pl.BlockSpec((blk, 1), lambda i: (i, 0)),
            pl.BlockSpec((D,), lambda i: (0,)),
        ],
        out_specs=pl.BlockSpec((blk, D), lambda i: (i, 0)),
        out_shape=jax.ShapeDtypeStruct((n, D), jnp.float32),
    )(scat, h, deg.reshape(n, 1), b)


CH = 16384
CHB = 14
CHT = CH // NS


def _cnt_body(nbg, nblk, dst_hbm, out_hbm, dbuf, hist):
    c = lax.axis_index("c")
    s = lax.axis_index("s")
    wid = s * NC + c
    zero = jnp.zeros((16,), jnp.int32)
    one = jnp.full((16,), 1, jnp.int32)
    for b in range(nbg):
        hist[pl.ds(b * 16, 16)] = zero

    def grp16(i, carry):
        bv = dbuf[pl.ds(i * 16, 16)] >> CHB
        for j in range(16):
            b16 = bv[j] * 16
            hist[pl.ds(b16, 16)] = hist[pl.ds(b16, 16)] + one
        return carry

    def blk(bi, carry):
        e0 = pl.multiple_of((wid * nblk + bi) * 512, 512)
        pltpu.sync_copy(dst_hbm.at[pl.ds(e0, 512)], dbuf)
        return lax.fori_loop(0, 32, grp16, carry)

    lax.fori_loop(0, nblk, blk, 0)
    pltpu.sync_copy(hist, out_hbm.at[pl.ds(pl.multiple_of(wid * nbg * 16, 16), nbg * 16)])


def _bin_body(nbg, nblk, offp, src_hbm, dst_hbm, offs_hbm,
              bsrc_hbm, bdst_hbm, sbuf, dbuf, offs_v, stg_s, stg_d,
              fill, pos):
    c = lax.axis_index("c")
    s = lax.axis_index("s")
    wid = s * NC + c
    zero = jnp.zeros((16,), jnp.int32)
    lanes = lax.iota(jnp.int32, 16)
    pltpu.sync_copy(offs_hbm, offs_v)
    for b in range(nbg):
        fill[pl.ds(b * 16, 16)] = zero
        pv = offs_v[pl.ds(b * 32 + wid, 16)]
        pos[pl.ds(b * 16, 16)] = jnp.full((16,), 1, jnp.int32) * pv[0]

    def grp16(i, carry):
            dv = dbuf[pl.ds(i * 16, 16)]
            sv = sbuf[pl.ds(i * 16, 16)]
            bv = dv >> CHB
            dl = dv & (CH - 1)
            for j in range(16):
                b16 = bv[j] * 16
                b144 = bv[j] * 144
                f = fill[pl.ds(b16, 16)][0]
                stg_s[pl.ds(b144 + f, 16)] = jnp.full((16,), 1, jnp.int32) * sv[j]
                stg_d[pl.ds(b144 + f, 16)] = jnp.full((16,), 1, jnp.int32) * dl[j]
                fnew = f + 1

                @pl.when(fnew == 128)
                def _fl():
                    p = pl.multiple_of(pos[pl.ds(b16, 16)][0], 128)
                    pltpu.sync_copy(stg_s.at[pl.ds(b144, 128)],
                                    bsrc_hbm.at[pl.ds(p, 128)])
                    pltpu.sync_copy(stg_d.at[pl.ds(b144, 128)],
                                    bdst_hbm.at[pl.ds(p, 128)])
                    pos[pl.ds(b16, 16)] = jnp.full((16,), 1, jnp.int32) * (p + 128)

                fill[pl.ds(b16, 16)] = (jnp.full((16,), 1, jnp.int32)
                                        * jnp.where(fnew == 128, 0, fnew))
            return carry

    def blk(bi, carry):
        e0 = pl.multiple_of((wid * nblk + bi) * 512, 512)
        pltpu.sync_copy(src_hbm.at[pl.ds(e0, 512)], sbuf)
        pltpu.sync_copy(dst_hbm.at[pl.ds(e0, 512)], dbuf)
        return lax.fori_loop(0, 32, grp16, carry)

    lax.fori_loop(0, nblk, blk, 0)
    for b in range(nbg):
        f = fill[pl.ds(b * 16, 16)][0]

        @pl.when(f > 0)
        def _tail():
            for i in range(8):
                ln = i * 16 + lanes
                cs = stg_s[pl.ds(b * 144 + i * 16, 16)]
                cd = stg_d[pl.ds(b * 144 + i * 16, 16)]
                keep = ln < f
                stg_s[pl.ds(b * 144 + i * 16, 16)] = jnp.where(keep, cs, 0)
                stg_d[pl.ds(b * 144 + i * 16, 16)] = jnp.where(keep, cd, CH)
            p = pl.multiple_of(pos[pl.ds(b * 16, 16)][0], 128)
            pltpu.sync_copy(stg_s.at[pl.ds(b * 144, 128)],
                            bsrc_hbm.at[pl.ds(p, 128)])
            pltpu.sync_copy(stg_d.at[pl.ds(b * 144, 128)],
                            bdst_hbm.at[pl.ds(p, 128)])


def _scat2_body(npass, offp, hs_hbm, bsrc_hbm, bdst_hbm, offs_hbm, zz_hbm,
                out_hbm, offs_v, gsrcA, gdstA, gsrcB, gdstB, addiA, addiB,
                rowsA, rowsB, semIA, semIB, semGA, semGB, semAA, semAB,
                shared):
    c = lax.axis_index("c")
    s = lax.axis_index("s")
    pltpu.sync_copy(offs_hbm, offs_v)

    def do_pass(p, carry):
        b = p * NC + c
        lo = b * CH
        rs = pl.multiple_of(offs_v[pl.ds(b * 32, 16)][0], 128)
        re = offs_v[pl.ds((b + 1) * 32, 16)][0]
        ng = (re - rs) >> 7
        woff = pl.multiple_of(s * CHT, 1024)
        pltpu.sync_copy(zz_hbm.at[pl.ds(woff, CHT)],
                        shared.at[pl.ds(woff, CHT)])
        plsc.subcore_barrier()
        g0 = (ng * s) >> 4
        g1 = (ng * (s + 1)) >> 4
        tn = g1 - g0

        def gbase(g):
            gc = jnp.minimum(g, g1 - 1)
            return pl.multiple_of(rs + gc * 128, 128)

        def idx_load(g, gsrc, gdst, sem):
            base = gbase(g)
            pltpu.async_copy(bsrc_hbm.at[pl.ds(base, 128)], gsrc, sem)
            pltpu.async_copy(bdst_hbm.at[pl.ds(base, 128)], gdst, sem)

        def idx_wait(gsrc, gdst, sem):
            pltpu.make_async_copy(bsrc_hbm.at[pl.ds(0, 128)], gsrc, sem).wait()
            pltpu.make_async_copy(bdst_hbm.at[pl.ds(0, 128)], gdst, sem).wait()

        def gather(g, gsrc, rows, sem):
            del g
            pltpu.async_copy(hs_hbm.at[gsrc], rows, sem)

        def gather_wait(rows, sem):
            pltpu.make_async_copy(hs_hbm.at[pl.ds(0, 128)], rows, sem).wait()

        def copy128(srcv, dstv):
            for i in range(8):
                dstv[pl.ds(i * 16, 16)] = srcv[pl.ds(i * 16, 16)]

        def add_async(rows, addi, sem):
            pltpu.async_copy(rows, shared.at[addi], sem, add=True)

        def add_wait(rows, sem):
            pltpu.make_async_copy(hs_hbm.at[pl.ds(0, 128)], rows, sem).wait()

        @pl.when(tn > 0)
        def _work():
            idx_load(g0, gsrcA, gdstA, semIA)
            idx_wait(gsrcA, gdstA, semIA)
            gather(g0, gsrcA, rowsA, semGA)
            idx_load(g0 + 1, gsrcB, gdstB, semIB)
            npair = tn >> 1

            def pair(q, cc):
                g = g0 + q * 2
                idx_wait(gsrcB, gdstB, semIB)

                @pl.when(q > 0)
                def _wb():
                    add_wait(rowsB, semAB)

                gather(g + 1, gsrcB, rowsB, semGB)
                gather_wait(rowsA, semGA)
                copy128(gdstA, addiA)
                add_async(rowsA, addiA, semAA)
                idx_load(g + 2, gsrcA, gdstA, semIA)
                idx_wait(gsrcA, gdstA, semIA)
                add_wait(rowsA, semAA)
                gather(g + 2, gsrcA, rowsA, semGA)
                gather_wait(rowsB, semGB)
                copy128(gdstB, addiB)
                add_async(rowsB, addiB, semAB)
                idx_load(g + 3, gsrcB, gdstB, semIB)
                return cc

            lax.fori_loop(0, npair, pair, 0)

            @pl.when(npair > 0)
            def _drainb():
                add_wait(rowsB, semAB)

            gather_wait(rowsA, semGA)

            @pl.when((tn & 1) == 1)
            def _odd():
                pltpu.sync_copy(rowsA, shared.at[gdstA], add=True)

            idx_wait(gsrcB, gdstB, semIB)

        plsc.subcore_barrier()
        pltpu.sync_copy(shared.at[pl.ds(woff, CHT)],
                        out_hbm.at[pl.ds(pl.multiple_of(lo + s * CHT, 1024), CHT)])
        plsc.subcore_barrier()
        return carry

    lax.fori_loop(0, npass, do_pass, 0)


def _mesh():
    return plsc.VectorSubcoreMesh(core_axis_name="c", subcore_axis_name="s")


def _graph_params(E, n):
    epad = -(-E // 16384) * 16384
    npass = -(-n // (NC * CH))
    nbg = npass * NC
    npad = nbg * CH
    sz = epad + 32 * nbg * 128
    offp = -(-(nbg * 32 + 16) // 8) * 8
    return epad, npass, nbg, npad, sz, offp


def _sc_prep(src, dst, E, n):
    epad, npass, nbg, npad, sz, offp = _graph_params(E, n)
    if epad != E:
        pad = epad - E
        src = jnp.concatenate([src, jnp.zeros((pad,), jnp.int32)])
        dst = jnp.concatenate([dst, jnp.full((pad,), npad - 1, jnp.int32)])
    nblk = epad // (32 * 512)
    cnt = pl.kernel(
        functools.partial(_cnt_body, nbg, nblk),
        out_type=jax.ShapeDtypeStruct((32 * nbg * 16,), jnp.int32),
        mesh=_mesh(),
        scratch_types=[
            pltpu.VMEM((512,), jnp.int32),
            pltpu.VMEM((nbg * 16,), jnp.int32),
        ],
    )(dst)
    counts = cnt.reshape(32, nbg, 16)[:, :, 0]
    pc = ((counts + 127) // 128) * 128
    flat = pc.T.reshape(-1)
    offs = jnp.concatenate([jnp.zeros((1,), jnp.int32),
                            jnp.cumsum(flat, dtype=jnp.int32)])
    offs = jnp.concatenate(
        [offs, jnp.full((offp - nbg * 32 - 1,), offs[-1], jnp.int32)])
    bsrc, bdst = pl.kernel(
        functools.partial(_bin_body, nbg, nblk, offp),
        out_type=[jax.ShapeDtypeStruct((sz,), jnp.int32),
                  jax.ShapeDtypeStruct((sz,), jnp.int32)],
        mesh=_mesh(),
        scratch_types=[
            pltpu.VMEM((512,), jnp.int32),
            pltpu.VMEM((512,), jnp.int32),
            pltpu.VMEM((offp,), jnp.int32),
            pltpu.VMEM((nbg * 144,), jnp.int32),
            pltpu.VMEM((nbg * 144,), jnp.int32),
            pltpu.VMEM((nbg * 16,), jnp.int32),
            pltpu.VMEM((nbg * 16,), jnp.int32),
        ],
    )(src, dst, offs)
    return bsrc, bdst, offs


def _sc_scatter2(hs, prep, E, n):
    bsrc, bdst, offs = prep
    epad, npass, nbg, npad, sz, offp = _graph_params(E, n)
    zz = jnp.zeros((CH, D), jnp.float32)
    out = pl.kernel(
        functools.partial(_scat2_body, npass, offp),
        out_type=jax.ShapeDtypeStruct((npad, D), jnp.float32),
        mesh=_mesh(),
        compiler_params=pltpu.CompilerParams(use_tc_tiling_on_sc=False),
        scratch_types=[
            pltpu.VMEM((offp,), jnp.int32),
            pltpu.VMEM((128,), jnp.int32),
            pltpu.VMEM((128,), jnp.int32),
            pltpu.VMEM((128,), jnp.int32),
            pltpu.VMEM((128,), jnp.int32),
            pltpu.VMEM((128,), jnp.int32),
            pltpu.VMEM((128,), jnp.int32),
            pltpu.VMEM((128, D), jnp.float32),
            pltpu.VMEM((128, D), jnp.float32),
            pltpu.SemaphoreType.DMA,
            pltpu.SemaphoreType.DMA,
            pltpu.SemaphoreType.DMA,
            pltpu.SemaphoreType.DMA,
            pltpu.SemaphoreType.DMA,
            pltpu.SemaphoreType.DMA,
            pltpu.VMEM_SHARED((CH + 8, D), jnp.float32),
        ],
    )(hs, bsrc, bdst, offs, zz)
    return out[:n]


def _gcn(x, prep, E, W, b, n, deg):
    h, hs = _tc_pre(x, W, deg)
    scat = _sc_scatter2(hs, prep, E, n)
    return _tc_post(scat, h, deg, b)


def kernel(Gu, Gi, Ge, Wpn0, bpn0, Wpn1, bpn1, Wpe0, bpe0, Wpe1, bpe1,
           Wnn0, bnn0, Wnn1, bnn1, Wee0, bee0, Wee1, bee1, Wne0, bne0,
           Wne1, bne1, edge_index, node_edge_index, edge_edge_index):
    nn_emb = jnp.concatenate([Gu, Gi], axis=0)
    ee_emb = Ge

    dst_all = jnp.concatenate([
        edge_index[1],
        edge_edge_index[1] + NN,
        node_edge_index[1] + (NN + NE),
        jnp.full((DEG_ROWS * 128 - E_TOT,), DEG_N, jnp.int32),
    ]).reshape(DEG_ROWS, 128)
    zz = jnp.zeros((DEG_SH,), jnp.float32)
    parts = _sc_degree(dst_all, zz).reshape(NC, DEG_SH)
    hist = parts[0] + parts[1]
    deg_nn = hist[0:NN] + 1.0
    deg_ee = hist[NN:NN + NE] + 1.0
    deg_ne = hist[NN + NE:NN + NE + NNE] + 1.0

    nn_proj = _tc_proj(nn_emb, Wpn0, bpn0, Wpn1, bpn1)
    ee_proj = _tc_proj(ee_emb, Wpe0, bpe0, Wpe1, bpe1)
    ne_emb = jnp.concatenate([nn_proj, ee_proj], axis=0)
    Wnn = [(Wnn0, bnn0), (Wnn1, bnn1)]
    Wee = [(Wee0, bee0), (Wee1, bee1)]
    Wne = [(Wne0, bne0), (Wne1, bne1)]
    prep_nn = _sc_prep(edge_index[0], edge_index[1], 800000, NN)
    prep_ee = _sc_prep(edge_edge_index[0], edge_edge_index[1], 800000, NE)
    prep_ne = _sc_prep(node_edge_index[0], node_edge_index[1], 1600000, NNE)
    for l in range(2):
        nn_emb = _gcn(nn_emb, prep_nn, 800000, Wnn[l][0], Wnn[l][1], NN, deg_nn)
        ee_emb = _gcn(ee_emb, prep_ee, 800000, Wee[l][0], Wee[l][1], NE, deg_ee)
        ne_emb = _gcn(ne_emb, prep_ne, 1600000, Wne[l][0], Wne[l][1], NNE, deg_ne)
        ne_node = ne_emb[:NN]
        ne_edge = ne_emb[NN:]
        nn_emb = jnp.concatenate([nn_emb, ne_node], axis=1)
        ee_emb = jnp.concatenate([ee_emb, ne_edge], axis=1)
        ne_emb = jnp.concatenate([nn_emb, ee_emb], axis=0)
    gu = nn_emb[:NU]
    gi = nn_emb[NU:]
    gu3 = gu.reshape(25, 1000, 2 * D)
    gi3 = gi.reshape(25, 1000, 2 * D)
    xui = pl.pallas_call(
        _dot_kernel,
        grid=(25,),
        in_specs=[
            pl.BlockSpec((1, 1000, 2 * D), lambda i: (i, 0, 0)),
            pl.BlockSpec((1, 1000, 2 * D), lambda i: (i, 0, 0)),
        ],
        out_specs=pl.BlockSpec((1, 1, 1000), lambda i: (i, 0, 0)),
        out_shape=jax.ShapeDtypeStruct((25, 1, 1000), jnp.float32),
    )(gu3, gi3)
    return xui.reshape(NU)
